# baseline (speedup 1.0000x reference)
import jax
import jax.numpy as jnp
from jax import lax
from jax.experimental import pallas as pl

NSTRIPE = 8
EPS = 1e-06


def _body(x_ref, o_ref):
    xb = x_ref[0]
    c, t, hw = xb.shape
    X2 = xb.reshape(c * t, hw)

    jio = lax.broadcasted_iota(jnp.int32, (hw, NSTRIPE), 0)
    sio = lax.broadcasted_iota(jnp.int32, (hw, NSTRIPE), 1)
    P = jnp.where(jio // 16 == sio, 1.0 / 16.0, 0.0).astype(jnp.float32)

    F = lax.dot(X2, P, precision=lax.Precision.HIGHEST)
    F3 = F.reshape(c, t, NSTRIPE)

    S = jnp.maximum(jnp.sum(F3 * F3, axis=0), EPS)

    tio = lax.broadcasted_iota(jnp.int32, (t, NSTRIPE), 0)
    m1 = jnp.max(S, axis=0)
    i1 = jnp.min(jnp.where(S == m1[None, :], tio, t), axis=0)
    Sm = jnp.where(tio == i1[None, :], -1.0, S)
    m2 = jnp.max(Sm, axis=0)
    i2 = jnp.min(jnp.where(Sm == m2[None, :], tio, t), axis=0)

    Wt = jnp.where((tio == i1[None, :]) | (tio == i2[None, :]), 0.5, 0.0)
    H = jnp.sum(F3 * Wt[None, :, :], axis=1)
    o_ref[0] = H.T


def kernel(x):
    n, c, t, h, w = x.shape
    xr = x.reshape(n, c, t, h * w)
    out = pl.pallas_call(
        _body,
        grid=(n,),
        in_specs=[pl.BlockSpec((1, c, t, h * w), lambda i: (i, 0, 0, 0))],
        out_specs=pl.BlockSpec((1, NSTRIPE, c), lambda i: (i, 0, 0)),
        out_shape=jax.ShapeDtypeStruct((n, NSTRIPE, c), jnp.float32),
    )(xr)
    return out.reshape(n, NSTRIPE * c)


# speedup vs baseline: 1.1972x; 1.1972x over previous
import jax
import jax.numpy as jnp
from jax import lax
from jax.experimental import pallas as pl

NSTRIPE = 8
EPS = 1e-06


def _body(x_ref, o_ref):
    xb = x_ref[0]
    c, t, hw = xb.shape
    X2 = xb.reshape(c * t, hw)

    jio = lax.broadcasted_iota(jnp.int32, (hw, NSTRIPE), 0)
    sio = lax.broadcasted_iota(jnp.int32, (hw, NSTRIPE), 1)
    P = jnp.where(jio // 16 == sio, 1.0 / 16.0, 0.0).astype(jnp.float32)

    Pb = P.astype(jnp.bfloat16)
    xh = X2.astype(jnp.bfloat16)
    r1 = X2 - xh.astype(jnp.float32)
    xl = r1.astype(jnp.bfloat16)
    xl2 = (r1 - xl.astype(jnp.float32)).astype(jnp.bfloat16)
    F = (lax.dot(xh, Pb, preferred_element_type=jnp.float32)
         + lax.dot(xl, Pb, preferred_element_type=jnp.float32)
         + lax.dot(xl2, Pb, preferred_element_type=jnp.float32))
    F3 = F.reshape(c, t, NSTRIPE)

    S = jnp.maximum(jnp.sum(F3 * F3, axis=0), EPS)

    tio = lax.broadcasted_iota(jnp.int32, (t, NSTRIPE), 0)
    m1 = jnp.max(S, axis=0)
    i1 = jnp.min(jnp.where(S == m1[None, :], tio, t), axis=0)
    Sm = jnp.where(tio == i1[None, :], -1.0, S)
    m2 = jnp.max(Sm, axis=0)
    i2 = jnp.min(jnp.where(Sm == m2[None, :], tio, t), axis=0)

    Wt = jnp.where((tio == i1[None, :]) | (tio == i2[None, :]), 0.5, 0.0)
    H = jnp.sum(F3 * Wt[None, :, :], axis=1)
    o_ref[0] = H.T


def kernel(x):
    n, c, t, h, w = x.shape
    xr = x.reshape(n, c, t, h * w)
    out = pl.pallas_call(
        _body,
        grid=(n,),
        in_specs=[pl.BlockSpec((1, c, t, h * w), lambda i: (i, 0, 0, 0))],
        out_specs=pl.BlockSpec((1, NSTRIPE, c), lambda i: (i, 0, 0)),
        out_shape=jax.ShapeDtypeStruct((n, NSTRIPE, c), jnp.float32),
    )(xr)
    return out.reshape(n, NSTRIPE * c)
